# baseline (speedup 1.0000x reference)
import functools

import jax
import jax.numpy as jnp
from jax.experimental import pallas as pl
from jax.experimental.pallas import tpu as pltpu

F = 1024
H = 4096
BM = 1024
HC = 512
NJ = H // HC
L1C = 1024


def _body(x_ref, w1_ref, b1_ref, w2_ref, b2_ref, w3_ref, b3_ref, o_ref,
          h1_ref):
    j = pl.program_id(1)

    @pl.when(j == 0)
    def _():
        xb = (2.0 * x_ref[...]).astype(jnp.bfloat16)
        for c in range(H // L1C):
            cols = slice(c * L1C, (c + 1) * L1C)
            h1 = jnp.dot(xb, w1_ref[:, cols], preferred_element_type=jnp.float32)
            h1_ref[:, cols] = jnp.maximum(h1 + b1_ref[:, cols], 0.0).astype(jnp.bfloat16)
        o_ref[...] = x_ref[...] + b3_ref[...]

    w2b = w2_ref[...].astype(jnp.bfloat16)
    for hf in range(2):
        rows = slice(hf * (BM // 2), (hf + 1) * (BM // 2))
        h2 = jnp.dot(h1_ref[rows, :], w2b,
                     preferred_element_type=jnp.float32)
        h2 = jnp.maximum(h2 + b2_ref[...], 0.0).astype(jnp.bfloat16)
        o_ref[rows, :] += jnp.dot(h2, w3_ref[...],
                                  preferred_element_type=jnp.float32)


@functools.partial(jax.jit, static_argnums=())
def kernel(morph0_features, W1, b1, W2, b2, W3, b3):
    B = morph0_features.shape[0]
    w1b = W1.astype(jnp.bfloat16)
    w3b = W3.astype(jnp.bfloat16)
    b1r = b1.reshape(1, H)
    b2r = b2.reshape(1, H)
    b3r = b3.reshape(1, F)

    grid = (B // BM, NJ)
    out = pl.pallas_call(
        _body,
        grid=grid,
        in_specs=[
            pl.BlockSpec((BM, F), lambda i, j: (i, 0)),
            pl.BlockSpec((F, H), lambda i, j: (0, 0)),
            pl.BlockSpec((1, H), lambda i, j: (0, 0)),
            pl.BlockSpec((H, HC), lambda i, j: (0, j)),
            pl.BlockSpec((1, HC), lambda i, j: (0, j)),
            pl.BlockSpec((HC, F), lambda i, j: (j, 0)),
            pl.BlockSpec((1, F), lambda i, j: (0, 0)),
        ],
        out_specs=pl.BlockSpec((BM, F), lambda i, j: (i, 0)),
        out_shape=jax.ShapeDtypeStruct((B, F), jnp.float32),
        scratch_shapes=[
            pltpu.VMEM((BM, H), jnp.bfloat16),
        ],
        compiler_params=pltpu.CompilerParams(
            dimension_semantics=("parallel", "arbitrary"),
        ),
    )(morph0_features, w1b, b1r, W2, b2r, w3b, b3r)
    return out


# speedup vs baseline: 1.0012x; 1.0012x over previous
import functools

import jax
import jax.numpy as jnp
from jax.experimental import pallas as pl
from jax.experimental.pallas import tpu as pltpu

F = 1024
H = 4096
BM = 1024
HC = 512
NJ = H // HC
L1C = 1024


def _body(x_ref, w1_ref, b1_ref, w2_ref, b2_ref, w3_ref, b3_ref, o_ref,
          h1_ref):
    j = pl.program_id(1)

    @pl.when(j == 0)
    def _():
        xb = (2.0 * x_ref[...]).astype(jnp.bfloat16)
        for c in range(H // L1C):
            cols = slice(c * L1C, (c + 1) * L1C)
            h1 = jnp.dot(xb, w1_ref[:, cols], preferred_element_type=jnp.float32)
            h1_ref[:, cols] = jnp.maximum(h1 + b1_ref[:, cols], 0.0).astype(jnp.bfloat16)
        o_ref[...] = x_ref[...] + b3_ref[...]

    w2b = w2_ref[...].astype(jnp.bfloat16)
    for hf in range(2):
        rows = slice(hf * (BM // 2), (hf + 1) * (BM // 2))
        h2 = jnp.dot(h1_ref[rows, :], w2b,
                     preferred_element_type=jnp.float32)
        h2 = jnp.maximum(h2 + b2_ref[...], 0.0).astype(jnp.bfloat16)
        o_ref[rows, :] += jnp.dot(h2, w3_ref[...],
                                  preferred_element_type=jnp.float32)


@functools.partial(jax.jit, static_argnums=())
def kernel(morph0_features, W1, b1, W2, b2, W3, b3):
    B = morph0_features.shape[0]
    w1b = W1.astype(jnp.bfloat16)
    w3b = W3.astype(jnp.bfloat16)
    b1r = b1.reshape(1, H)
    b2r = b2.reshape(1, H)
    b3r = b3.reshape(1, F)

    grid = (B // BM, NJ)
    out = pl.pallas_call(
        _body,
        grid=grid,
        in_specs=[
            pl.BlockSpec((BM, F), lambda i, j: (i, 0)),
            pl.BlockSpec((F, H), lambda i, j: (0, 0)),
            pl.BlockSpec((1, H), lambda i, j: (0, 0)),
            pl.BlockSpec((H, HC), lambda i, j: (0, j)),
            pl.BlockSpec((1, HC), lambda i, j: (0, j)),
            pl.BlockSpec((HC, F), lambda i, j: (j, 0)),
            pl.BlockSpec((1, F), lambda i, j: (0, 0)),
        ],
        out_specs=pl.BlockSpec((BM, F), lambda i, j: (i, 0)),
        out_shape=jax.ShapeDtypeStruct((B, F), jnp.float32),
        scratch_shapes=[
            pltpu.VMEM((BM, H), jnp.bfloat16),
        ],
        compiler_params=pltpu.CompilerParams(
            dimension_semantics=("arbitrary", "arbitrary"),
        ),
    )(morph0_features, w1b, b1r, W2, b2r, w3b, b3r)
    return out
